# Initial kernel scaffold; baseline (speedup 1.0000x reference)
import jax
import jax.numpy as jnp
from jax.experimental import pallas as pl


def kernel(feat, edge_index, W_neigh, b_neigh, W_self):
    raise NotImplementedError("write your pallas kernel here")


# speedup vs baseline: 5.3845x; 5.3845x over previous
import functools

import jax
import jax.numpy as jnp
from jax import lax
from jax.experimental import pallas as pl
from jax.experimental.pallas import tpu as pltpu
from jax.experimental.pallas import tpu_sc as plsc

N = 10000
E = 320000
D = 128
NC = 2
NS = 16
NW = NC * NS
EPW = E // NW
K = 80
ITERS = EPW // K
NP = 10240
RPT = NP // NS
ZR = 32


def _sc_gather_scatter(y, src, dst):
    mesh = plsc.VectorSubcoreMesh(core_axis_name="c", subcore_axis_name="s")

    @functools.partial(
        pl.kernel,
        out_type=jax.ShapeDtypeStruct((NC, NP, D), jnp.float32),
        mesh=mesh,
        scratch_types=[
            pltpu.VMEM((K,), jnp.int32),
            pltpu.VMEM((K,), jnp.int32),
            pltpu.VMEM((K, D), jnp.float32),
            pltpu.VMEM((ZR, D), jnp.float32),
            pltpu.VMEM_SHARED((NP, D), jnp.float32),
            pltpu.SemaphoreType.DMA,
        ],
    )
    def k(y_hbm, src_hbm, dst_hbm, out_hbm, sidx, didx, rows, zbuf, accum, sem):
        c = lax.axis_index("c")
        s = lax.axis_index("s")
        wid = c * NS + s

        z = jnp.zeros((16,), jnp.float32)

        def zrow(i, _):
            for j in range(D // 16):
                zbuf[i, pl.ds(j * 16, 16)] = z
            return 0

        lax.fori_loop(0, ZR, zrow, 0)
        r0 = s * RPT

        def zslab(i, _):
            pltpu.sync_copy(zbuf, accum.at[pl.ds(r0 + i * ZR, ZR)])
            return 0

        lax.fori_loop(0, RPT // ZR, zslab, 0)
        plsc.subcore_barrier()

        e0 = wid * EPW

        def body(i, _):
            b = e0 + i * K
            pltpu.sync_copy(src_hbm.at[pl.ds(b, K)], sidx)
            pltpu.sync_copy(dst_hbm.at[pl.ds(b, K)], didx)
            pltpu.async_copy(y_hbm.at[sidx], rows, sem).wait()
            pltpu.sync_copy(rows, accum.at[didx], add=True)
            return 0

        lax.fori_loop(0, ITERS, body, 0)
        plsc.subcore_barrier()

        pltpu.sync_copy(accum.at[pl.ds(r0, RPT)], out_hbm.at[c, pl.ds(r0, RPT)])

    return k(y, src, dst)


def _tc_prep(feat, w_neigh, w_self, b_neigh):
    bn = 1000

    def body(f_ref, wn_ref, ws_ref, b_ref, y_ref, base_ref):
        f = f_ref[...]
        y_ref[...] = jnp.dot(f, wn_ref[...], preferred_element_type=jnp.float32)
        base_ref[...] = (
            jnp.dot(f, ws_ref[...], preferred_element_type=jnp.float32) + b_ref[...]
        )

    return pl.pallas_call(
        body,
        grid=(N // bn,),
        in_specs=[
            pl.BlockSpec((bn, D), lambda i: (i, 0)),
            pl.BlockSpec((D, D), lambda i: (0, 0)),
            pl.BlockSpec((D, D), lambda i: (0, 0)),
            pl.BlockSpec((1, D), lambda i: (0, 0)),
        ],
        out_specs=[
            pl.BlockSpec((bn, D), lambda i: (i, 0)),
            pl.BlockSpec((bn, D), lambda i: (i, 0)),
        ],
        out_shape=[
            jax.ShapeDtypeStruct((N, D), jnp.float32),
            jax.ShapeDtypeStruct((N, D), jnp.float32),
        ],
    )(feat, w_neigh, w_self, b_neigh.reshape(1, D))


def _tc_combine(partials, base):
    bn = 1000

    def body(p_ref, b_ref, o_ref):
        o_ref[...] = p_ref[0] + p_ref[1] + b_ref[...]

    return pl.pallas_call(
        body,
        grid=(N // bn,),
        in_specs=[
            pl.BlockSpec((NC, bn, D), lambda i: (0, i, 0)),
            pl.BlockSpec((bn, D), lambda i: (i, 0)),
        ],
        out_specs=pl.BlockSpec((bn, D), lambda i: (i, 0)),
        out_shape=jax.ShapeDtypeStruct((N, D), jnp.float32),
    )(partials, base)


def kernel(feat, edge_index, W_neigh, b_neigh, W_self):
    src = edge_index[0]
    dst = edge_index[1]
    y, base = _tc_prep(feat, W_neigh, W_self, b_neigh)
    partials = _sc_gather_scatter(y, src, dst)
    return _tc_combine(partials, base)


# speedup vs baseline: 10.7335x; 1.9934x over previous
import functools

import jax
import jax.numpy as jnp
from jax import lax
from jax.experimental import pallas as pl
from jax.experimental.pallas import tpu as pltpu
from jax.experimental.pallas import tpu_sc as plsc

N = 10000
E = 320000
D = 128
NC = 2
NS = 16
NW = NC * NS
EPW = E // NW
K = 80
ITERS = EPW // K
NSTAGE = 5
IPS = ITERS // NSTAGE
NP = 10240
RPT = NP // NS


def _sc_gather_scatter(y, src, dst):
    mesh = plsc.VectorSubcoreMesh(core_axis_name="c", subcore_axis_name="s")

    @functools.partial(
        pl.kernel,
        out_type=jax.ShapeDtypeStruct((NC, NP, D), jnp.float32),
        mesh=mesh,
        scratch_types=[
            pltpu.VMEM((IPS, K), jnp.int32),
            pltpu.VMEM((IPS, K), jnp.int32),
            pltpu.VMEM((K, D), jnp.float32),
            pltpu.VMEM((K, D), jnp.float32),
            pltpu.VMEM_SHARED((NP, D), jnp.float32),
            pltpu.SemaphoreType.DMA,
            pltpu.SemaphoreType.DMA,
        ],
    )
    def k(y_hbm, src_hbm, dst_hbm, out_hbm, sidx, didx, rows0, rows1,
          accum, sem0, sem1):
        c = lax.axis_index("c")
        s = lax.axis_index("s")
        wid = c * NS + s

        z = jnp.zeros((16,), jnp.float32)

        def zrow(i, _):
            for j in range(D // 16):
                rows0[i, pl.ds(j * 16, 16)] = z
            return 0

        lax.fori_loop(0, K, zrow, 0)
        r0 = s * RPT
        for j in range(RPT // K):
            pltpu.sync_copy(rows0, accum.at[pl.ds(r0 + j * K, K)])
        plsc.subcore_barrier()

        for sg in range(NSTAGE):
            pltpu.sync_copy(src_hbm.at[wid, sg], sidx)
            pltpu.sync_copy(dst_hbm.at[wid, sg], didx)
            pltpu.async_copy(y_hbm.at[sidx.at[0]], rows0, sem0)

            def body(i, _):
                c0 = 2 * i
                pltpu.async_copy(y_hbm.at[sidx.at[c0 + 1]], rows1, sem1)
                pltpu.make_async_copy(y_hbm.at[sidx.at[c0]], rows0, sem0).wait()
                pltpu.sync_copy(rows0, accum.at[didx.at[c0]], add=True)
                pltpu.async_copy(y_hbm.at[sidx.at[c0 + 2]], rows0, sem0)
                pltpu.make_async_copy(y_hbm.at[sidx.at[c0 + 1]], rows1, sem1).wait()
                pltpu.sync_copy(rows1, accum.at[didx.at[c0 + 1]], add=True)
                return 0

            lax.fori_loop(0, (IPS - 1) // 2, body, 0)
            pltpu.make_async_copy(y_hbm.at[sidx.at[IPS - 1]], rows0, sem0).wait()
            pltpu.sync_copy(rows0, accum.at[didx.at[IPS - 1]], add=True)

        plsc.subcore_barrier()

        pltpu.sync_copy(accum.at[pl.ds(r0, RPT)], out_hbm.at[c, pl.ds(r0, RPT)])

    return k(y, src.reshape(NW, NSTAGE, IPS, K), dst.reshape(NW, NSTAGE, IPS, K))


def _tc_prep(feat, w_neigh, w_self, b_neigh):
    bn = 1000

    def body(f_ref, wn_ref, ws_ref, b_ref, y_ref, base_ref):
        f = f_ref[...]
        y_ref[...] = jnp.dot(f, wn_ref[...], preferred_element_type=jnp.float32)
        base_ref[...] = (
            jnp.dot(f, ws_ref[...], preferred_element_type=jnp.float32) + b_ref[...]
        )

    return pl.pallas_call(
        body,
        grid=(N // bn,),
        in_specs=[
            pl.BlockSpec((bn, D), lambda i: (i, 0)),
            pl.BlockSpec((D, D), lambda i: (0, 0)),
            pl.BlockSpec((D, D), lambda i: (0, 0)),
            pl.BlockSpec((1, D), lambda i: (0, 0)),
        ],
        out_specs=[
            pl.BlockSpec((bn, D), lambda i: (i, 0)),
            pl.BlockSpec((bn, D), lambda i: (i, 0)),
        ],
        out_shape=[
            jax.ShapeDtypeStruct((N, D), jnp.float32),
            jax.ShapeDtypeStruct((N, D), jnp.float32),
        ],
    )(feat, w_neigh, w_self, b_neigh.reshape(1, D))


def _tc_combine(partials, base):
    bn = 1000

    def body(p_ref, b_ref, o_ref):
        o_ref[...] = p_ref[0] + p_ref[1] + b_ref[...]

    return pl.pallas_call(
        body,
        grid=(N // bn,),
        in_specs=[
            pl.BlockSpec((NC, bn, D), lambda i: (0, i, 0)),
            pl.BlockSpec((bn, D), lambda i: (i, 0)),
        ],
        out_specs=pl.BlockSpec((bn, D), lambda i: (i, 0)),
        out_shape=jax.ShapeDtypeStruct((N, D), jnp.float32),
    )(partials, base)


def kernel(feat, edge_index, W_neigh, b_neigh, W_self):
    src = edge_index[0]
    dst = edge_index[1]
    y, base = _tc_prep(feat, W_neigh, W_self, b_neigh)
    partials = _sc_gather_scatter(y, src, dst)
    return _tc_combine(partials, base)
